# Initial kernel scaffold; baseline (speedup 1.0000x reference)
import jax
import jax.numpy as jnp
from jax.experimental import pallas as pl


def kernel(pos, gbparameters, edge_index, batch):
    raise NotImplementedError("write your pallas kernel here")


# speedup vs baseline: 35.9200x; 35.9200x over previous
import functools

import jax
import jax.numpy as jnp
from jax import lax
from jax.experimental import pallas as pl
from jax.experimental.pallas import tpu as pltpu
from jax.experimental.pallas import tpu_sc as plsc

N = 50000
E = 1600000
NMOL = 100
ATOMS = 500
NP = 50176
S16 = NP // 16
CH = 128
NCHUNK = E // CH
NG = CH // 16
CPT = (NCHUNK + 31) // 32
OFFSET = 0.0195141
PREFAC = -138.935485 * (1.0 - 1.0 / 78.5)
C1, C2, C3 = 1.09511284, 1.907992938, 2.50798245
LN2 = 0.6931471805599453


def _mesh():
    return plsc.VectorSubcoreMesh(
        core_axis_name="c", subcore_axis_name="s", num_cores=2,
        num_subcores=16)


_SC_PARAMS = pltpu.CompilerParams(needs_layout_passes=False)


def _rsqrt(x):
    i = lax.bitcast_convert_type(x, jnp.int32)
    y = lax.bitcast_convert_type(jnp.int32(0x5F3759DF) - (i >> 1), jnp.float32)
    for _ in range(3):
        y = y * (1.5 - 0.5 * x * y * y)
    return y


def _log(x):
    i = lax.bitcast_convert_type(x, jnp.int32)
    e = (i >> 23) - 127
    m = lax.bitcast_convert_type(
        (i & jnp.int32(0x007FFFFF)) | jnp.int32(0x3F800000), jnp.float32)
    big = m > 1.4142135
    m = jnp.where(big, 0.5 * m, m)
    e = (e + jnp.where(big, jnp.int32(1), jnp.int32(0))).astype(jnp.float32)
    t = (m - 1.0) / (m + 1.0)
    t2 = t * t
    p = 2.0 * t * (1.0 + t2 * (1.0 / 3.0 + t2 * (0.2 + t2 * (1.0 / 7.0))))
    return e * LN2 + p


def _tanh(x):
    a = jnp.abs(x)
    th = 1.0 - 2.0 / (jnp.exp(2.0 * a) + 1.0)
    return jnp.sign(x) * th


def _rem500(v):
    return lax.rem(v, jnp.full((16,), ATOMS, jnp.int32))


def _ids():
    cid = lax.axis_index("c")
    sid = lax.axis_index("s")
    return cid, sid, cid * 16 + sid


def _zero_spmem(zbuf, refs, offs):
    @pl.loop(0, S16 // 16)
    def _z(i):
        zbuf[pl.ds(i * 16, 16)] = jnp.zeros((16,), jnp.float32)
    for r in refs:
        pltpu.sync_copy(zbuf, r.at[pl.ds(offs, S16)])


def _pass_a(src, dst, px, py, pz, tr, ts,
            outI, outd, outx, outy, outz,
            ps_x, ps_y, ps_z, acc_I,
            sbuf, dbuf, gxs, gys, gzs, gxd, gyd, gzd,
            od, ox, oy, oz, oI, tab_r, tab_s, zbuf, sem):
    cid, sid, wid = _ids()
    offs = sid * S16
    pltpu.sync_copy(tr, tab_r)
    pltpu.sync_copy(ts, tab_s)
    for hsrc, shr in ((px, ps_x), (py, ps_y), (pz, ps_z)):
        pltpu.sync_copy(hsrc.at[pl.ds(offs, S16)], zbuf)
        pltpu.sync_copy(zbuf, shr.at[pl.ds(offs, S16)])
    _zero_spmem(zbuf, [acc_I], offs)
    plsc.subcore_barrier()

    @pl.loop(0, CPT)
    def _chunk(j):
        g = j * 32 + wid

        @pl.when(g < NCHUNK)
        def _():
            off = g * CH
            pltpu.sync_copy(src.at[pl.ds(off, CH)], sbuf)
            pltpu.sync_copy(dst.at[pl.ds(off, CH)], dbuf)
            cps = [pltpu.async_copy(ps_x.at[sbuf], gxs, sem),
                   pltpu.async_copy(ps_y.at[sbuf], gys, sem),
                   pltpu.async_copy(ps_z.at[sbuf], gzs, sem),
                   pltpu.async_copy(ps_x.at[dbuf], gxd, sem),
                   pltpu.async_copy(ps_y.at[dbuf], gyd, sem),
                   pltpu.async_copy(ps_z.at[dbuf], gzd, sem)]
            for c in cps:
                c.wait()
            for gi in range(NG):
                sl = pl.ds(gi * 16, 16)
                sv = sbuf[sl]
                dv = dbuf[sl]
                dx = gxs[sl] - gxd[sl]
                dy = gys[sl] - gyd[sl]
                dz = gzs[sl] - gzd[sl]
                d2 = dx * dx + dy * dy + dz * dz + 1e-12
                invd = _rsqrt(d2)
                d = d2 * invd
                r_i = plsc.load_gather(tab_r, [_rem500(dv)])
                s = plsc.load_gather(tab_s, [_rem500(sv)])
                U = d + s
                invU = 1.0 / U
                dms = d - s
                adms = jnp.abs(dms)
                L = jnp.where(adms < r_i, r_i, adms)
                invL = 1.0 / L
                logLU = _log(L * invU)
                A = d - s * s * invd
                I = 0.5 * (invL - invU
                           + 0.25 * A * (invU * invU - invL * invL)
                           + 0.5 * logLU * invd)
                I = jnp.where(r_i < U, I, 0.0)
                od[sl] = d
                ox[sl] = dx
                oy[sl] = dy
                oz[sl] = dz
                oI[sl] = I
            pltpu.sync_copy(od, outd.at[pl.ds(off, CH)])
            pltpu.sync_copy(ox, outx.at[pl.ds(off, CH)])
            pltpu.sync_copy(oy, outy.at[pl.ds(off, CH)])
            pltpu.sync_copy(oz, outz.at[pl.ds(off, CH)])
            pltpu.sync_copy(oI, acc_I.at[dbuf], add=True)

    plsc.subcore_barrier()
    pltpu.sync_copy(acc_I.at[pl.ds(offs, S16)], zbuf)
    pltpu.sync_copy(zbuf, outI.at[pl.ds(cid * NP + offs, S16)])


def _pass_c(src, dst, dh, Ipart, tq, trho,
            outB, outdBdI, outes, outdbs, outpair, outdEdB,
            B_s, acc_pair, acc_dEdB,
            sbuf, dbuf, dcb, oe, obi, obj,
            nb0, nb1, nb2, nb3, B_v, tab_q, tab_rho, sem):
    cid, sid, wid = _ids()
    offs = sid * S16
    pltpu.sync_copy(tq, tab_q)
    pltpu.sync_copy(trho, tab_rho)
    _zero_spmem(nb0, [acc_pair, acc_dEdB], offs)
    pltpu.sync_copy(Ipart.at[pl.ds(offs, S16)], nb0)
    pltpu.sync_copy(Ipart.at[pl.ds(NP + offs, S16)], nb1)

    @pl.loop(0, S16 // 16)
    def _node(i):
        sl = pl.ds(i * 16, 16)
        n = offs + i * 16 + lax.iota(jnp.int32, 16)
        a = _rem500(n)
        rho = plsc.load_gather(tab_rho, [a])
        rho_off = rho - OFFSET
        Isum = nb0[sl] + nb1[sl]
        psi = Isum * rho_off
        T = psi * (C1 + psi * (-C2 + psi * C3))
        th = _tanh(T)
        Binv = 1.0 / rho_off - th / rho
        B = 1.0 / Binv
        dT = C1 + psi * (-2.0 * C2 + psi * (3.0 * C3))
        dBdI = B * B * (1.0 - th * th) / rho * dT * rho_off
        q = plsc.load_gather(tab_q, [a])
        es = 0.5 * PREFAC * q * q * Binv
        dbs = -0.5 * PREFAC * q * q * Binv * Binv
        nb0[sl] = B
        nb1[sl] = dBdI
        nb2[sl] = es
        nb3[sl] = dbs

    pltpu.sync_copy(nb0, B_s.at[pl.ds(offs, S16)])

    @pl.when(cid == 0)
    def _():
        pltpu.sync_copy(nb0, outB.at[pl.ds(offs, S16)])
        pltpu.sync_copy(nb1, outdBdI.at[pl.ds(offs, S16)])
        pltpu.sync_copy(nb2, outes.at[pl.ds(offs, S16)])
        pltpu.sync_copy(nb3, outdbs.at[pl.ds(offs, S16)])

    plsc.subcore_barrier()
    pltpu.sync_copy(B_s, B_v)

    @pl.loop(0, CPT)
    def _chunk(j):
        g = j * 32 + wid

        @pl.when(g < NCHUNK)
        def _():
            off = g * CH
            pltpu.sync_copy(src.at[pl.ds(off, CH)], sbuf)
            pltpu.sync_copy(dst.at[pl.ds(off, CH)], dbuf)
            pltpu.sync_copy(dh.at[pl.ds(off, CH)], dcb)
            for gi in range(NG):
                sl = pl.ds(gi * 16, 16)
                sv = sbuf[sl]
                dv = dbuf[sl]
                d = dcb[sl]
                Bi = plsc.load_gather(B_v, [dv])
                Bj = plsc.load_gather(B_v, [sv])
                qi = plsc.load_gather(tab_q, [_rem500(dv)])
                qj = plsc.load_gather(tab_q, [_rem500(sv)])
                P = Bi * Bj
                invP = 1.0 / P
                gg = jnp.exp(-0.25 * d * d * invP)
                f2 = d * d + P * gg
                invf = _rsqrt(f2)
                K = (0.5 * PREFAC) * qi * qj
                common = (-K * invf * invf * invf) * (0.5 * gg) \
                    * (1.0 + 0.25 * d * d * invP)
                oe[sl] = K * invf
                obi[sl] = common * Bj
                obj[sl] = common * Bi
            pltpu.sync_copy(oe, acc_pair.at[dbuf], add=True)
            pltpu.sync_copy(obi, acc_dEdB.at[dbuf], add=True)
            pltpu.sync_copy(obj, acc_dEdB.at[sbuf], add=True)

    plsc.subcore_barrier()
    pltpu.sync_copy(acc_pair.at[pl.ds(offs, S16)], nb0)
    pltpu.sync_copy(nb0, outpair.at[pl.ds(cid * NP + offs, S16)])
    pltpu.sync_copy(acc_dEdB.at[pl.ds(offs, S16)], nb1)
    pltpu.sync_copy(nb1, outdEdB.at[pl.ds(cid * NP + offs, S16)])


def _pass_e(src, dst, dh, dxh, dyh, dzh, Bh, dEdB, dbsh, dBdIh, tq, tr, ts,
            outgx, outgy, outgz,
            acc_gx, acc_gy, acc_gz, dEdI_s,
            sbuf, dbuf, dcb, dxb, dyb, dzb,
            ovx, ovy, ovz, onx, ony, onz,
            nb0, nb1, nb2, nb3, B_v, dEdI_v, tab_q, tab_r, tab_s, sem):
    cid, sid, wid = _ids()
    offs = sid * S16
    pltpu.sync_copy(tq, tab_q)
    pltpu.sync_copy(tr, tab_r)
    pltpu.sync_copy(ts, tab_s)
    _zero_spmem(nb0, [acc_gx, acc_gy, acc_gz], offs)
    pltpu.sync_copy(dEdB.at[pl.ds(offs, S16)], nb0)
    pltpu.sync_copy(dEdB.at[pl.ds(NP + offs, S16)], nb1)
    pltpu.sync_copy(dbsh.at[pl.ds(offs, S16)], nb2)
    pltpu.sync_copy(dBdIh.at[pl.ds(offs, S16)], nb3)

    @pl.loop(0, S16 // 16)
    def _node(i):
        sl = pl.ds(i * 16, 16)
        nb0[sl] = (nb0[sl] + nb1[sl] + nb2[sl]) * nb3[sl]

    pltpu.sync_copy(nb0, dEdI_s.at[pl.ds(offs, S16)])
    plsc.subcore_barrier()
    pltpu.sync_copy(Bh, B_v)
    pltpu.sync_copy(dEdI_s, dEdI_v)

    @pl.loop(0, CPT)
    def _chunk(j):
        g = j * 32 + wid

        @pl.when(g < NCHUNK)
        def _():
            off = g * CH
            pltpu.sync_copy(src.at[pl.ds(off, CH)], sbuf)
            pltpu.sync_copy(dst.at[pl.ds(off, CH)], dbuf)
            pltpu.sync_copy(dh.at[pl.ds(off, CH)], dcb)
            pltpu.sync_copy(dxh.at[pl.ds(off, CH)], dxb)
            pltpu.sync_copy(dyh.at[pl.ds(off, CH)], dyb)
            pltpu.sync_copy(dzh.at[pl.ds(off, CH)], dzb)
            for gi in range(NG):
                sl = pl.ds(gi * 16, 16)
                sv = sbuf[sl]
                dv = dbuf[sl]
                d = dcb[sl]
                invd = 1.0 / d
                r_i = plsc.load_gather(tab_r, [_rem500(dv)])
                s = plsc.load_gather(tab_s, [_rem500(sv)])
                U = d + s
                invU = 1.0 / U
                dms = d - s
                adms = jnp.abs(dms)
                use_r = adms < r_i
                L = jnp.where(use_r, r_i, adms)
                invL = 1.0 / L
                logLU = _log(L * invU)
                A = d - s * s * invd
                dA = 1.0 + s * s * invd * invd
                Lp = jnp.where(use_r, 0.0, jnp.sign(dms))
                invL2 = invL * invL
                invU2 = invU * invU
                dI = 0.5 * (-Lp * invL2 + invU2
                            + 0.25 * dA * (invU2 - invL2)
                            + 0.25 * A * (-2.0 * invU2 * invU
                                          + 2.0 * Lp * invL2 * invL)
                            + 0.5 * ((Lp * invL - invU) * invd
                                     - logLU * invd * invd))
                dI = jnp.where(r_i < U, dI, 0.0)
                Bi = plsc.load_gather(B_v, [dv])
                Bj = plsc.load_gather(B_v, [sv])
                qi = plsc.load_gather(tab_q, [_rem500(dv)])
                qj = plsc.load_gather(tab_q, [_rem500(sv)])
                P = Bi * Bj
                invP = 1.0 / P
                gg = jnp.exp(-0.25 * d * d * invP)
                f2 = d * d + P * gg
                invf = _rsqrt(f2)
                K = (0.5 * PREFAC) * qi * qj
                ded = (-K * invf * invf * invf) * d * (1.0 - 0.25 * gg)
                dEdIv = plsc.load_gather(dEdI_v, [dv])
                coef = (ded + dEdIv * dI) * invd
                vx = coef * dxb[sl]
                vy = coef * dyb[sl]
                vz = coef * dzb[sl]
                ovx[sl] = vx
                ovy[sl] = vy
                ovz[sl] = vz
                onx[sl] = -vx
                ony[sl] = -vy
                onz[sl] = -vz
            pltpu.sync_copy(ovx, acc_gx.at[sbuf], add=True)
            pltpu.sync_copy(ovy, acc_gy.at[sbuf], add=True)
            pltpu.sync_copy(ovz, acc_gz.at[sbuf], add=True)
            pltpu.sync_copy(onx, acc_gx.at[dbuf], add=True)
            pltpu.sync_copy(ony, acc_gy.at[dbuf], add=True)
            pltpu.sync_copy(onz, acc_gz.at[dbuf], add=True)

    plsc.subcore_barrier()
    for shr, hbm in ((acc_gx, outgx), (acc_gy, outgy), (acc_gz, outgz)):
        pltpu.sync_copy(shr.at[pl.ds(offs, S16)], nb0)
        pltpu.sync_copy(nb0, hbm.at[pl.ds(cid * NP + offs, S16)])


def _tc_forces(gx, gy, gz, fx, fy, fz):
    fx[...] = -(gx[0] + gx[1])
    fy[...] = -(gy[0] + gy[1])
    fz[...] = -(gz[0] + gz[1])


def _tc_emol(p0, p1, es, out):
    e = p0[...] + p1[...] + es[...]
    s = jnp.sum(e, axis=1, keepdims=True)
    out[...] = jnp.broadcast_to(s, out.shape)


def kernel(pos, gbparameters, edge_index, batch):
    f32 = jnp.float32
    src = edge_index[0].astype(jnp.int32)
    dst = edge_index[1].astype(jnp.int32)
    padn = NP - N
    px = jnp.pad(pos[:, 0].astype(f32), (0, padn))
    py = jnp.pad(pos[:, 1].astype(f32), (0, padn))
    pz = jnp.pad(pos[:, 2].astype(f32), (0, padn))
    q = gbparameters[:, 0].astype(f32)
    rho = gbparameters[:, 1].astype(f32)
    scale = gbparameters[:, 2].astype(f32)
    tq = jnp.pad(q, (0, 512 - ATOMS))
    trho = jnp.pad(rho, (0, 512 - ATOMS), constant_values=1.0)
    tr = jnp.pad(rho - OFFSET, (0, 512 - ATOMS), constant_values=1.0)
    ts = jnp.pad(scale * (rho - OFFSET), (0, 512 - ATOMS),
                 constant_values=1.0)

    sdt = jax.ShapeDtypeStruct
    pass_a = pl.kernel(
        _pass_a,
        compiler_params=_SC_PARAMS,
        out_type=(sdt((2 * NP,), f32), sdt((E,), f32), sdt((E,), f32),
                  sdt((E,), f32), sdt((E,), f32)),
        mesh=_mesh(),
        scratch_types=(
            [pltpu.VMEM_SHARED((NP,), f32)] * 4
            + [pltpu.VMEM((CH,), jnp.int32)] * 2
            + [pltpu.VMEM((CH,), f32)] * 11
            + [pltpu.VMEM((512,), f32)] * 2
            + [pltpu.VMEM((S16,), f32), pltpu.SemaphoreType.DMA]),
    )
    Ipart, dh, dxh, dyh, dzh = pass_a(src, dst, px, py, pz, tr, ts)

    pass_c = pl.kernel(
        _pass_c,
        compiler_params=_SC_PARAMS,
        out_type=(sdt((NP,), f32), sdt((NP,), f32), sdt((NP,), f32),
                  sdt((NP,), f32), sdt((2 * NP,), f32), sdt((2 * NP,), f32)),
        mesh=_mesh(),
        scratch_types=(
            [pltpu.VMEM_SHARED((NP,), f32)] * 3
            + [pltpu.VMEM((CH,), jnp.int32)] * 2
            + [pltpu.VMEM((CH,), f32)] * 4
            + [pltpu.VMEM((S16,), f32)] * 4
            + [pltpu.VMEM((NP,), f32)]
            + [pltpu.VMEM((512,), f32)] * 2
            + [pltpu.SemaphoreType.DMA]),
    )
    Bh, dBdIh, esh, dbsh, pair, dEdB = pass_c(src, dst, dh, Ipart, tq, trho)

    pass_e = pl.kernel(
        _pass_e,
        compiler_params=_SC_PARAMS,
        out_type=(sdt((2 * NP,), f32), sdt((2 * NP,), f32), sdt((2 * NP,), f32)),
        mesh=_mesh(),
        scratch_types=(
            [pltpu.VMEM_SHARED((NP,), f32)] * 4
            + [pltpu.VMEM((CH,), jnp.int32)] * 2
            + [pltpu.VMEM((CH,), f32)] * 10
            + [pltpu.VMEM((S16,), f32)] * 4
            + [pltpu.VMEM((NP,), f32)] * 2
            + [pltpu.VMEM((512,), f32)] * 3
            + [pltpu.SemaphoreType.DMA]),
    )
    gx, gy, gz = pass_e(src, dst, dh, dxh, dyh, dzh, Bh, dEdB, dbsh, dBdIh,
                        tq, tr, ts)

    g3 = lambda a: a.reshape(2, NP // 128, 128)
    fx, fy, fz = pl.pallas_call(
        _tc_forces,
        out_shape=(sdt((NP // 128, 128), f32),) * 3,
    )(g3(gx), g3(gy), g3(gz))
    forces = jnp.stack(
        [fx.reshape(NP)[:N], fy.reshape(NP)[:N], fz.reshape(NP)[:N]], axis=1)

    seg = lambda a: jnp.pad(a[:N].reshape(NMOL, ATOMS), ((0, 4), (0, 12)))
    emol = pl.pallas_call(
        _tc_emol,
        out_shape=sdt((NMOL + 4, 128), f32),
    )(seg(pair[:NP]), seg(pair[NP:]), seg(esh))
    energy = emol[:NMOL, 0:1]

    return (energy, forces)
